# baseline (speedup 1.0000x reference)
import functools

import jax
import jax.numpy as jnp
from jax import lax
from jax.experimental import pallas as pl
from jax.experimental.pallas import tpu as pltpu
from jax.experimental.pallas import tpu_sc as plsc

N_NODES = 10000
NP = 10240
E = 320000
D = 128
CH = 128
NC, NS = 2, 16
NW = NC * NS
CPT = 80
NCHP = NW * CPT
EPAD = NCHP * CH
RPT = NP // NS
NB = 2
NQ = 2
HCPT = CPT // NQ

_MESH = plsc.VectorSubcoreMesh(core_axis_name="c", subcore_axis_name="s")
_SC_PARAMS = pltpu.CompilerParams(needs_layout_passes=False)


def _worker_id():
    cid = lax.axis_index("c")
    sid = lax.axis_index("s")
    return cid, sid, sid * NC + cid


def _degree_body(sidx_hbm, ridx_hbm, out_hbm, sh, rh, sbuf, rbuf):
    _, _, wid = _worker_id()

    def zero_body(i, _):
        sh[pl.ds(i * 16, 16)] = jnp.zeros((16,), jnp.float32)
        rh[pl.ds(i * 16, 16)] = jnp.zeros((16,), jnp.float32)
        return 0

    lax.fori_loop(0, NP // 16, zero_body, 0)

    pltpu.sync_copy(sidx_hbm.at[pl.ds(wid * CPT, CPT)], sbuf)
    pltpu.sync_copy(ridx_hbm.at[pl.ds(wid * CPT, CPT)], rbuf)

    ones = jnp.ones((16,), jnp.float32)

    def chunk_body(j, _):
        for k in range(CH // 16):
            plsc.addupdate_scatter(sh, [sbuf[j, pl.ds(k * 16, 16)]], ones)
            plsc.addupdate_scatter(rh, [rbuf[j, pl.ds(k * 16, 16)]], ones)
        return 0

    lax.fori_loop(0, CPT, chunk_body, 0)

    pltpu.sync_copy(sh, out_hbm.at[0, wid])
    pltpu.sync_copy(rh, out_hbm.at[1, wid])


_degree_call = pl.kernel(
    _degree_body,
    out_type=jax.ShapeDtypeStruct((2, NW, NP), jnp.float32),
    mesh=_MESH,
    compiler_params=_SC_PARAMS,
    scratch_types=[
        pltpu.VMEM((NP,), jnp.float32),
        pltpu.VMEM((NP,), jnp.float32),
        pltpu.VMEM((CPT, CH), jnp.int32),
        pltpu.VMEM((CPT, CH), jnp.int32),
    ],
)


BM = 10240


def _mm_body(x_ref, w_ref, b_ref, out_ref):
    out_ref[...] = jnp.dot(x_ref[...], w_ref[...],
                           preferred_element_type=jnp.float32) + b_ref[...]


def _mm_call(x, W, b):
    return pl.pallas_call(
        _mm_body,
        grid=(NP // BM,),
        in_specs=[
            pl.BlockSpec((BM, D), lambda m: (m, 0)),
            pl.BlockSpec((D, D), lambda m: (0, 0)),
            pl.BlockSpec((D,), lambda m: (0,)),
        ],
        out_specs=pl.BlockSpec((BM, D), lambda m: (m, 0)),
        out_shape=jax.ShapeDtypeStruct((NP, D), jnp.float32),
    )(x, W, b)


def _scale_body(n_ref, degs_ref, out_ref):
    sdeg = jnp.sum(degs_ref[0], axis=0)
    sinv = lax.rsqrt(jnp.maximum(sdeg, 1.0))
    out_ref[...] = n_ref[...] * sinv[:, None]


def _scale_call(nodes, degs):
    return pl.pallas_call(
        _scale_body,
        grid=(NP // BM,),
        in_specs=[
            pl.BlockSpec((BM, D), lambda m: (m, 0)),
            pl.BlockSpec((1, NW, BM), lambda m: (0, 0, m)),
        ],
        out_specs=pl.BlockSpec((BM, D), lambda m: (m, 0)),
        out_shape=jax.ShapeDtypeStruct((NP, D), jnp.float32),
    )(nodes, degs)


def _agg_body(table_hbm, sidx_hbm, ridx_hbm, out_hbm,
              sidx_v, ridx_v, rows_v, gsem, ssem, acc):
    cid, sid, wid = _worker_id()

    def zrow_body(r, _):
        for k in range(D // 16):
            rows_v[0, r, pl.ds(k * 16, 16)] = jnp.zeros((16,), jnp.float32)
        return 0

    lax.fori_loop(0, CH, zrow_body, 0)
    for i in range(RPT // CH):
        pltpu.sync_copy(rows_v.at[0], acc.at[pl.ds(sid * RPT + i * CH, CH)])
    plsc.subcore_barrier()

    for h in range(NQ):
        base = wid * CPT + h * HCPT
        pltpu.sync_copy(sidx_hbm.at[pl.ds(base, HCPT)], sidx_v)
        pltpu.sync_copy(ridx_hbm.at[pl.ds(base, HCPT)], ridx_v)

        for b in range(NB):
            pltpu.async_copy(table_hbm.at[sidx_v.at[b]], rows_v.at[b],
                             gsem.at[b])

        def pipe_body(q, _):
            for b in range(NB):
                j = q * NB + b
                pltpu.make_async_copy(table_hbm.at[sidx_v.at[j]],
                                      rows_v.at[b], gsem.at[b]).wait()
                pltpu.sync_copy(rows_v.at[b], acc.at[ridx_v.at[j]], add=True)
                nxt = j + NB

                @pl.when(nxt < HCPT)
                def _():
                    pltpu.async_copy(table_hbm.at[sidx_v.at[nxt]],
                                     rows_v.at[b], gsem.at[b])
            return 0

        lax.fori_loop(0, HCPT // NB, pipe_body, 0)
    plsc.subcore_barrier()

    pltpu.sync_copy(acc.at[pl.ds(sid * RPT, RPT)],
                    out_hbm.at[cid, pl.ds(sid * RPT, RPT)])


_agg_call = pl.kernel(
    _agg_body,
    out_type=jax.ShapeDtypeStruct((NC, NP, D), jnp.float32),
    mesh=_MESH,
    compiler_params=_SC_PARAMS,
    scratch_types=[
        pltpu.VMEM((HCPT, CH), jnp.int32),
        pltpu.VMEM((HCPT, CH), jnp.int32),
        pltpu.VMEM((NB, CH, D), jnp.float32),
        pltpu.SemaphoreType.DMA((NB,)),
        pltpu.SemaphoreType.DMA((NB,)),
        pltpu.VMEM_SHARED((NP, D), jnp.float32),
    ],
)


def _fin_body(p_ref, degs_ref, o_ref):
    s = p_ref[0] + p_ref[1]
    rdeg = jnp.sum(degs_ref[0], axis=0)
    rinv = lax.rsqrt(jnp.maximum(rdeg, 1.0))
    o_ref[...] = s * rinv[:, None]


def _fin_call(partial, degs):
    return pl.pallas_call(
        _fin_body,
        grid=(NP // BM,),
        in_specs=[
            pl.BlockSpec((NC, BM, D), lambda m: (0, m, 0)),
            pl.BlockSpec((1, NW, BM), lambda m: (1, 0, m)),
        ],
        out_specs=pl.BlockSpec((BM, D), lambda m: (m, 0)),
        out_shape=jax.ShapeDtypeStruct((N_NODES, D), jnp.float32),
    )(partial, degs)


def kernel(x, edge_index, W, b):
    pad = jnp.tile(N_NODES + (jnp.arange(EPAD - E, dtype=jnp.int32)
                              % (NP - N_NODES))[None, :], (2, 1))
    ei = jnp.concatenate([edge_index, pad], axis=1)
    senders = ei[0].reshape(NCHP, CH)
    receivers = ei[1].reshape(NCHP, CH)

    degs = _degree_call(senders, receivers)
    nodes = _mm_call(x, W, b)
    table = _scale_call(nodes, degs)
    partial = _agg_call(table, senders, receivers)
    return _fin_call(partial, degs)


# speedup vs baseline: 1.0243x; 1.0243x over previous
import functools

import jax
import jax.numpy as jnp
from jax import lax
from jax.experimental import pallas as pl
from jax.experimental.pallas import tpu as pltpu
from jax.experimental.pallas import tpu_sc as plsc

N_NODES = 10000
NP = 10240
E = 320000
D = 128
CH = 128
NC, NS = 2, 16
NW = NC * NS
CPT = 80
NCHP = NW * CPT
EPAD = NCHP * CH
RPT = NP // NS
NB = 2
NQ = 2
HCPT = CPT // NQ

_MESH = plsc.VectorSubcoreMesh(core_axis_name="c", subcore_axis_name="s")
_SC_PARAMS = pltpu.CompilerParams(needs_layout_passes=False)


def _worker_id():
    cid = lax.axis_index("c")
    sid = lax.axis_index("s")
    return cid, sid, sid * NC + cid


def _degree_body(sidx_hbm, ridx_hbm, out_hbm, sh, rh, sbuf, rbuf):
    _, _, wid = _worker_id()

    def zero_body(i, _):
        sh[pl.ds(i * 16, 16)] = jnp.zeros((16,), jnp.float32)
        rh[pl.ds(i * 16, 16)] = jnp.zeros((16,), jnp.float32)
        return 0

    lax.fori_loop(0, NP // 16, zero_body, 0)

    pltpu.sync_copy(sidx_hbm.at[pl.ds(wid * CPT, CPT)], sbuf)
    pltpu.sync_copy(ridx_hbm.at[pl.ds(wid * CPT, CPT)], rbuf)

    ones = jnp.ones((16,), jnp.float32)

    def chunk_body(j, _):
        for k in range(CH // 16):
            plsc.addupdate_scatter(sh, [sbuf[j, pl.ds(k * 16, 16)]], ones)
            plsc.addupdate_scatter(rh, [rbuf[j, pl.ds(k * 16, 16)]], ones)
        return 0

    lax.fori_loop(0, CPT, chunk_body, 0)

    pltpu.sync_copy(sh, out_hbm.at[0, wid])
    pltpu.sync_copy(rh, out_hbm.at[1, wid])


_degree_call = pl.kernel(
    _degree_body,
    out_type=jax.ShapeDtypeStruct((2, NW, NP), jnp.float32),
    mesh=_MESH,
    compiler_params=_SC_PARAMS,
    scratch_types=[
        pltpu.VMEM((NP,), jnp.float32),
        pltpu.VMEM((NP,), jnp.float32),
        pltpu.VMEM((CPT, CH), jnp.int32),
        pltpu.VMEM((CPT, CH), jnp.int32),
    ],
)


BM = 5120


def _mm_body(x_ref, w_ref, b_ref, out_ref):
    out_ref[...] = jnp.dot(x_ref[...], w_ref[...],
                           preferred_element_type=jnp.float32) + b_ref[...]


def _mm_call(x, W, b):
    return pl.pallas_call(
        _mm_body,
        grid=(NP // BM,),
        in_specs=[
            pl.BlockSpec((BM, D), lambda m: (m, 0)),
            pl.BlockSpec((D, D), lambda m: (0, 0)),
            pl.BlockSpec((D,), lambda m: (0,)),
        ],
        out_specs=pl.BlockSpec((BM, D), lambda m: (m, 0)),
        out_shape=jax.ShapeDtypeStruct((NP, D), jnp.float32),
    )(x, W, b)


def _scale_body(n_ref, degs_ref, out_ref):
    sdeg = jnp.sum(degs_ref[0], axis=0)
    sinv = lax.rsqrt(jnp.maximum(sdeg, 1.0))
    out_ref[...] = n_ref[...] * sinv[:, None]


def _scale_call(nodes, degs):
    return pl.pallas_call(
        _scale_body,
        grid=(NP // BM,),
        in_specs=[
            pl.BlockSpec((BM, D), lambda m: (m, 0)),
            pl.BlockSpec((1, NW, BM), lambda m: (0, 0, m)),
        ],
        out_specs=pl.BlockSpec((BM, D), lambda m: (m, 0)),
        out_shape=jax.ShapeDtypeStruct((NP, D), jnp.float32),
    )(nodes, degs)


def _agg_body(table_hbm, sidx_hbm, ridx_hbm, out_hbm,
              sidx_v, ridx_v, rows_v, gsem, ssem, acc):
    cid, sid, wid = _worker_id()

    def zrow_body(r, _):
        for k in range(D // 16):
            rows_v[0, r, pl.ds(k * 16, 16)] = jnp.zeros((16,), jnp.float32)
        return 0

    lax.fori_loop(0, CH, zrow_body, 0)
    for i in range(RPT // CH):
        pltpu.sync_copy(rows_v.at[0], acc.at[pl.ds(sid * RPT + i * CH, CH)])
    plsc.subcore_barrier()

    for h in range(NQ):
        base = wid * CPT + h * HCPT
        pltpu.sync_copy(sidx_hbm.at[pl.ds(base, HCPT)], sidx_v)
        pltpu.sync_copy(ridx_hbm.at[pl.ds(base, HCPT)], ridx_v)

        for b in range(NB):
            pltpu.async_copy(table_hbm.at[sidx_v.at[b]], rows_v.at[b],
                             gsem.at[b])

        def pipe_body(q, _):
            for b in range(NB):
                j = q * NB + b
                pltpu.make_async_copy(table_hbm.at[sidx_v.at[j]],
                                      rows_v.at[b], gsem.at[b]).wait()
                pltpu.sync_copy(rows_v.at[b], acc.at[ridx_v.at[j]], add=True)
                nxt = j + NB

                @pl.when(nxt < HCPT)
                def _():
                    pltpu.async_copy(table_hbm.at[sidx_v.at[nxt]],
                                     rows_v.at[b], gsem.at[b])
            return 0

        lax.fori_loop(0, HCPT // NB, pipe_body, 0)
    plsc.subcore_barrier()

    pltpu.sync_copy(acc.at[pl.ds(sid * RPT, RPT)],
                    out_hbm.at[cid, pl.ds(sid * RPT, RPT)])


_agg_call = pl.kernel(
    _agg_body,
    out_type=jax.ShapeDtypeStruct((NC, NP, D), jnp.float32),
    mesh=_MESH,
    compiler_params=_SC_PARAMS,
    scratch_types=[
        pltpu.VMEM((HCPT, CH), jnp.int32),
        pltpu.VMEM((HCPT, CH), jnp.int32),
        pltpu.VMEM((NB, CH, D), jnp.float32),
        pltpu.SemaphoreType.DMA((NB,)),
        pltpu.SemaphoreType.DMA((NB,)),
        pltpu.VMEM_SHARED((NP, D), jnp.float32),
    ],
)


def _fin_body(p_ref, degs_ref, o_ref):
    s = p_ref[0] + p_ref[1]
    rdeg = jnp.sum(degs_ref[0], axis=0)
    rinv = lax.rsqrt(jnp.maximum(rdeg, 1.0))
    o_ref[...] = s * rinv[:, None]


def _fin_call(partial, degs):
    return pl.pallas_call(
        _fin_body,
        grid=(NP // BM,),
        in_specs=[
            pl.BlockSpec((NC, BM, D), lambda m: (0, m, 0)),
            pl.BlockSpec((1, NW, BM), lambda m: (1, 0, m)),
        ],
        out_specs=pl.BlockSpec((BM, D), lambda m: (m, 0)),
        out_shape=jax.ShapeDtypeStruct((N_NODES, D), jnp.float32),
    )(partial, degs)


def kernel(x, edge_index, W, b):
    pad = jnp.tile(N_NODES + (jnp.arange(EPAD - E, dtype=jnp.int32)
                              % (NP - N_NODES))[None, :], (2, 1))
    ei = jnp.concatenate([edge_index, pad], axis=1)
    senders = ei[0].reshape(NCHP, CH)
    receivers = ei[1].reshape(NCHP, CH)

    degs = _degree_call(senders, receivers)
    nodes = _mm_call(x, W, b)
    table = _scale_call(nodes, degs)
    partial = _agg_call(table, senders, receivers)
    return _fin_call(partial, degs)


# speedup vs baseline: 1.0248x; 1.0005x over previous
import jax
import jax.numpy as jnp
from jax import lax
from jax.experimental import pallas as pl
from jax.experimental.pallas import tpu as pltpu
from jax.experimental.pallas import tpu_sc as plsc

N_NODES = 10000
NP = 10240
E = 320000
D = 128
CH = 128
NC, NS = 2, 16
NW = NC * NS
CPT = 80
NCHP = NW * CPT
EPAD = NCHP * CH
RPT = NP // NS
NB = 2
NQ = 2
HCPT = CPT // NQ

_MESH = plsc.VectorSubcoreMesh(core_axis_name="c", subcore_axis_name="s")
_SC_PARAMS = pltpu.CompilerParams(needs_layout_passes=False)


def _worker_id():
    cid = lax.axis_index("c")
    sid = lax.axis_index("s")
    return cid, sid, sid * NC + cid


def _degree_body(sidx_hbm, ridx_hbm, out_hbm, sh, rh, sbuf, rbuf):
    _, _, wid = _worker_id()

    def zero_body(i, _):
        sh[pl.ds(i * 16, 16)] = jnp.zeros((16,), jnp.float32)
        rh[pl.ds(i * 16, 16)] = jnp.zeros((16,), jnp.float32)
        return 0

    lax.fori_loop(0, NP // 16, zero_body, 0)

    pltpu.sync_copy(sidx_hbm.at[pl.ds(wid * CPT, CPT)], sbuf)
    pltpu.sync_copy(ridx_hbm.at[pl.ds(wid * CPT, CPT)], rbuf)

    ones = jnp.ones((16,), jnp.float32)

    def chunk_body(j, _):
        for k in range(CH // 16):
            plsc.addupdate_scatter(sh, [sbuf[j, pl.ds(k * 16, 16)]], ones)
            plsc.addupdate_scatter(rh, [rbuf[j, pl.ds(k * 16, 16)]], ones)
        return 0

    lax.fori_loop(0, CPT, chunk_body, 0)

    pltpu.sync_copy(sh, out_hbm.at[0, wid])
    pltpu.sync_copy(rh, out_hbm.at[1, wid])


_degree_call = pl.kernel(
    _degree_body,
    out_type=jax.ShapeDtypeStruct((2, NW, NP), jnp.float32),
    mesh=_MESH,
    compiler_params=_SC_PARAMS,
    scratch_types=[
        pltpu.VMEM((NP,), jnp.float32),
        pltpu.VMEM((NP,), jnp.float32),
        pltpu.VMEM((CPT, CH), jnp.int32),
        pltpu.VMEM((CPT, CH), jnp.int32),
    ],
)


BM = 5120


def _mm_body(x_ref, w_ref, b_ref, out_ref):
    out_ref[...] = jnp.dot(x_ref[...], w_ref[...],
                           preferred_element_type=jnp.float32) + b_ref[...]


def _mm_call(x, W, b):
    return pl.pallas_call(
        _mm_body,
        grid=(NP // BM,),
        in_specs=[
            pl.BlockSpec((BM, D), lambda m: (m, 0)),
            pl.BlockSpec((D, D), lambda m: (0, 0)),
            pl.BlockSpec((D,), lambda m: (0,)),
        ],
        out_specs=pl.BlockSpec((BM, D), lambda m: (m, 0)),
        out_shape=jax.ShapeDtypeStruct((NP, D), jnp.float32),
    )(x, W, b)


def _scale_body(n_ref, degs_ref, out_ref):
    sdeg = jnp.sum(degs_ref[0], axis=0)
    sinv = lax.rsqrt(jnp.maximum(sdeg, 1.0))
    out_ref[...] = n_ref[...] * sinv[:, None]


def _scale_call(nodes, degs):
    return pl.pallas_call(
        _scale_body,
        grid=(NP // BM,),
        in_specs=[
            pl.BlockSpec((BM, D), lambda m: (m, 0)),
            pl.BlockSpec((1, NW, BM), lambda m: (0, 0, m)),
        ],
        out_specs=pl.BlockSpec((BM, D), lambda m: (m, 0)),
        out_shape=jax.ShapeDtypeStruct((NP, D), jnp.float32),
    )(nodes, degs)


def _agg_body(table_hbm, sidx_hbm, ridx_hbm, out_hbm,
              sidx_v, ridx_v, rows_v, gsem, ssem, acc):
    cid, sid, wid = _worker_id()

    def zrow_body(r, _):
        for k in range(D // 16):
            rows_v[0, r, pl.ds(k * 16, 16)] = jnp.zeros((16,), jnp.float32)
        return 0

    lax.fori_loop(0, CH, zrow_body, 0)
    for i in range(RPT // CH):
        pltpu.sync_copy(rows_v.at[0], acc.at[pl.ds(sid * RPT + i * CH, CH)])
    plsc.subcore_barrier()

    for h in range(NQ):
        base = wid * CPT + h * HCPT
        pltpu.sync_copy(sidx_hbm.at[pl.ds(base, HCPT)], sidx_v)
        pltpu.sync_copy(ridx_hbm.at[pl.ds(base, HCPT)], ridx_v)

        for b in range(NB):
            pltpu.async_copy(table_hbm.at[sidx_v.at[b]], rows_v.at[b],
                             gsem.at[b])

        def pipe_body(q, _):
            for b in range(NB):
                j = q * NB + b
                pltpu.make_async_copy(table_hbm.at[sidx_v.at[j]],
                                      rows_v.at[b], gsem.at[b]).wait()
                pltpu.sync_copy(rows_v.at[b], acc.at[ridx_v.at[j]], add=True)
                nxt = j + NB

                @pl.when(nxt < HCPT)
                def _():
                    pltpu.async_copy(table_hbm.at[sidx_v.at[nxt]],
                                     rows_v.at[b], gsem.at[b])
            return 0

        lax.fori_loop(0, HCPT // NB, pipe_body, 0)
    plsc.subcore_barrier()

    pltpu.sync_copy(acc.at[pl.ds(sid * RPT, RPT)],
                    out_hbm.at[cid, pl.ds(sid * RPT, RPT)])


_agg_call = pl.kernel(
    _agg_body,
    out_type=jax.ShapeDtypeStruct((NC, NP, D), jnp.float32),
    mesh=_MESH,
    compiler_params=_SC_PARAMS,
    scratch_types=[
        pltpu.VMEM((HCPT, CH), jnp.int32),
        pltpu.VMEM((HCPT, CH), jnp.int32),
        pltpu.VMEM((NB, CH, D), jnp.float32),
        pltpu.SemaphoreType.DMA((NB,)),
        pltpu.SemaphoreType.DMA((NB,)),
        pltpu.VMEM_SHARED((NP, D), jnp.float32),
    ],
)


def _fin_body(p_ref, degs_ref, o_ref):
    s = p_ref[0] + p_ref[1]
    rdeg = jnp.sum(degs_ref[0], axis=0)
    rinv = lax.rsqrt(jnp.maximum(rdeg, 1.0))
    o_ref[...] = s * rinv[:, None]


def _fin_call(partial, degs):
    return pl.pallas_call(
        _fin_body,
        grid=(NP // BM,),
        in_specs=[
            pl.BlockSpec((NC, BM, D), lambda m: (0, m, 0)),
            pl.BlockSpec((1, NW, BM), lambda m: (1, 0, m)),
        ],
        out_specs=pl.BlockSpec((BM, D), lambda m: (m, 0)),
        out_shape=jax.ShapeDtypeStruct((N_NODES, D), jnp.float32),
    )(partial, degs)


def kernel(x, edge_index, W, b):
    pad = jnp.tile(N_NODES + (jnp.arange(EPAD - E, dtype=jnp.int32)
                              % (NP - N_NODES))[None, :], (2, 1))
    ei = jnp.concatenate([edge_index, pad], axis=1)
    senders = ei[0].reshape(NCHP, CH)
    receivers = ei[1].reshape(NCHP, CH)

    degs = _degree_call(senders, receivers)
    nodes = _mm_call(x, W, b)
    table = _scale_call(nodes, degs)
    partial = _agg_call(table, senders, receivers)
    return _fin_call(partial, degs)
